# baseline (speedup 1.0000x reference)
import functools

import jax
import jax.numpy as jnp
from jax.experimental import pallas as pl
from jax.experimental.pallas import tpu as pltpu

_BIG = 9999999.0


def _mine_kernel(e1_ref, e2_ref, l1_ref, l2_ref, out_ref, inv2_ref, n1_ref):
    i = pl.program_id(0)
    j = pl.program_id(1)
    bm = e1_ref.shape[0]
    bn = e2_ref.shape[0]

    e2 = e2_ref[...]

    @pl.when(i == 0)
    def _cache_inv2():
        n2 = jnp.sqrt(jnp.sum(e2 * e2, axis=1, keepdims=True)) + 1e-12
        inv2_ref[pl.ds(j * bn, bn), :] = 1.0 / n2

    @pl.when(j == 0)
    def _cache_n1():
        e1 = e1_ref[...]
        n1 = jnp.sqrt(jnp.sum(e1 * e1, axis=1, keepdims=True)) + 1e-12
        n1_ref[0, pl.ds(i * bm, bm)] = n1.reshape(bm)

    raw = jax.lax.dot_general(
        e2, e1_ref[...], (((1,), (1,)), ((), ())),
        preferred_element_type=jnp.float32,
    )
    scaled = raw * inv2_ref[pl.ds(j * bn, bn), :]

    norm1 = n1_ref[0, pl.ds(i * bm, bm)].reshape(1, bm)
    mask = l2_ref[...] == l1_ref[...]
    p_sent = _BIG * norm1
    p_tile = jnp.min(jnp.where(mask, scaled, p_sent), axis=0) / norm1[0, :]
    n_tile = jnp.max(jnp.where(mask, -p_sent, scaled), axis=0) / norm1[0, :]

    @pl.when(j == 0)
    def _init():
        out_ref[0, :] = p_tile
        out_ref[1, :] = n_tile

    @pl.when(j != 0)
    def _fold():
        out_ref[0, :] = jnp.minimum(out_ref[0, :], p_tile)
        out_ref[1, :] = jnp.maximum(out_ref[1, :], n_tile)


@functools.partial(jax.jit, static_argnames=("bm", "bn"))
def _mine(emb1, emb2, label1, label2, bm=4096, bn=512):
    m, k = emb1.shape
    n = emb2.shape[0]
    l1 = label1.reshape(1, m)
    l2 = label2.reshape(n, 1)
    grid = (m // bm, n // bn)
    return pl.pallas_call(
        _mine_kernel,
        grid=grid,
        in_specs=[
            pl.BlockSpec((bm, k), lambda i, j: (i, 0)),
            pl.BlockSpec((bn, k), lambda i, j: (j, 0)),
            pl.BlockSpec((1, bm), lambda i, j: (0, i)),
            pl.BlockSpec((bn, 1), lambda i, j: (j, 0)),
        ],
        out_specs=pl.BlockSpec((2, bm), lambda i, j: (0, i)),
        out_shape=jax.ShapeDtypeStruct((2, m), jnp.float32),
        scratch_shapes=[
            pltpu.VMEM((n, 1), jnp.float32),
            pltpu.VMEM((1, m), jnp.float32),
        ],
        compiler_params=pltpu.CompilerParams(
            dimension_semantics=("arbitrary", "arbitrary"),
        ),
    )(emb1, emb2, l1, l2)


def kernel(emb1, emb2, label1, label2):
    return _mine(emb1, emb2, label1, label2)


# speedup vs baseline: 1.0044x; 1.0044x over previous
import functools

import jax
import jax.numpy as jnp
from jax.experimental import pallas as pl
from jax.experimental.pallas import tpu as pltpu

_BIG = 9999999.0


def _mine_kernel(e1_ref, e2_ref, l1_ref, l2_ref, out_ref, inv2_ref, n1_ref):
    i = pl.program_id(0)
    j = pl.program_id(1)
    bm = e1_ref.shape[0]
    bn = e2_ref.shape[0]

    e2 = e2_ref[...]

    @pl.when(i == 0)
    def _cache_inv2():
        n2 = jnp.sqrt(jnp.sum(e2 * e2, axis=1, keepdims=True)) + 1e-12
        inv2_ref[pl.ds(j * bn, bn), :] = 1.0 / n2

    @pl.when(j == 0)
    def _cache_n1():
        e1 = e1_ref[...]
        n1 = jnp.sqrt(jnp.sum(e1 * e1, axis=1, keepdims=True)) + 1e-12
        n1_ref[0, pl.ds(i * bm, bm)] = n1.reshape(bm)

    raw = jax.lax.dot_general(
        e2, e1_ref[...], (((1,), (1,)), ((), ())),
        preferred_element_type=jnp.float32,
    )
    scaled = raw * inv2_ref[pl.ds(j * bn, bn), :]

    norm1 = n1_ref[0, pl.ds(i * bm, bm)].reshape(1, bm)
    mask = l2_ref[...] == l1_ref[...]
    p_sent = _BIG * norm1
    p_tile = jnp.min(jnp.where(mask, scaled, p_sent), axis=0) / norm1[0, :]
    n_tile = jnp.max(jnp.where(mask, -p_sent, scaled), axis=0) / norm1[0, :]

    @pl.when(j == 0)
    def _init():
        out_ref[0, :] = p_tile
        out_ref[1, :] = n_tile

    @pl.when(j != 0)
    def _fold():
        out_ref[0, :] = jnp.minimum(out_ref[0, :], p_tile)
        out_ref[1, :] = jnp.maximum(out_ref[1, :], n_tile)


@functools.partial(jax.jit, static_argnames=("bm", "bn"))
def _mine(emb1, emb2, label1, label2, bm=4096, bn=2048):
    m, k = emb1.shape
    n = emb2.shape[0]
    l1 = label1.reshape(1, m)
    l2 = label2.reshape(n, 1)
    grid = (m // bm, n // bn)
    return pl.pallas_call(
        _mine_kernel,
        grid=grid,
        in_specs=[
            pl.BlockSpec((bm, k), lambda i, j: (i, 0)),
            pl.BlockSpec((bn, k), lambda i, j: (j, 0)),
            pl.BlockSpec((1, bm), lambda i, j: (0, i)),
            pl.BlockSpec((bn, 1), lambda i, j: (j, 0)),
        ],
        out_specs=pl.BlockSpec((2, bm), lambda i, j: (0, i)),
        out_shape=jax.ShapeDtypeStruct((2, m), jnp.float32),
        scratch_shapes=[
            pltpu.VMEM((n, 1), jnp.float32),
            pltpu.VMEM((1, m), jnp.float32),
        ],
        compiler_params=pltpu.CompilerParams(
            dimension_semantics=("arbitrary", "arbitrary"),
        ),
    )(emb1, emb2, l1, l2)


def kernel(emb1, emb2, label1, label2):
    return _mine(emb1, emb2, label1, label2)


# speedup vs baseline: 1.1623x; 1.1572x over previous
import functools

import jax
import jax.numpy as jnp
from jax.experimental import pallas as pl
from jax.experimental.pallas import tpu as pltpu

_BIG = 9999999.0


def _mine_kernel(e1_ref, e2_ref, l1_ref, l2_ref, out_ref, n1_ref):
    j = pl.program_id(0)
    bm = e1_ref.shape[0]

    @pl.when(j == 0)
    def _cache_n1():
        e1 = e1_ref[...]
        n1 = jnp.sqrt(jnp.sum(e1 * e1, axis=1, keepdims=True)) + 1e-12
        n1_ref[0, :] = n1.reshape(bm)

    e2 = e2_ref[...]
    inv2 = 1.0 / (jnp.sqrt(jnp.sum(e2 * e2, axis=1, keepdims=True)) + 1e-12)
    e2n = e2 * inv2

    raw = jax.lax.dot_general(
        e2n, e1_ref[...], (((1,), (1,)), ((), ())),
        preferred_element_type=jnp.float32,
    )

    norm1 = n1_ref[0, :].reshape(1, bm)
    mask = l2_ref[...] == l1_ref[...]
    p_sent = _BIG * norm1
    p_tile = jnp.min(jnp.where(mask, raw, p_sent), axis=0) / norm1[0, :]
    n_tile = jnp.max(jnp.where(mask, -p_sent, raw), axis=0) / norm1[0, :]

    @pl.when(j == 0)
    def _init():
        out_ref[0, :] = p_tile
        out_ref[1, :] = n_tile

    @pl.when(j != 0)
    def _fold():
        out_ref[0, :] = jnp.minimum(out_ref[0, :], p_tile)
        out_ref[1, :] = jnp.maximum(out_ref[1, :], n_tile)


@functools.partial(jax.jit, static_argnames=("bn",))
def _mine(emb1, emb2, label1, label2, bn=1024):
    m, k = emb1.shape
    n = emb2.shape[0]
    l1 = label1.reshape(1, m)
    l2 = label2.reshape(n, 1)
    grid = (n // bn,)
    return pl.pallas_call(
        _mine_kernel,
        grid=grid,
        in_specs=[
            pl.BlockSpec((m, k), lambda j: (0, 0)),
            pl.BlockSpec((bn, k), lambda j: (j, 0)),
            pl.BlockSpec((1, m), lambda j: (0, 0)),
            pl.BlockSpec((bn, 1), lambda j: (j, 0)),
        ],
        out_specs=pl.BlockSpec((2, m), lambda j: (0, 0)),
        out_shape=jax.ShapeDtypeStruct((2, m), jnp.float32),
        scratch_shapes=[pltpu.VMEM((1, m), jnp.float32)],
        compiler_params=pltpu.CompilerParams(
            dimension_semantics=("arbitrary",),
        ),
    )(emb1, emb2, l1, l2)


def kernel(emb1, emb2, label1, label2):
    return _mine(emb1, emb2, label1, label2)


# speedup vs baseline: 1.1682x; 1.0051x over previous
import functools

import jax
import jax.numpy as jnp
from jax.experimental import pallas as pl
from jax.experimental.pallas import tpu as pltpu

_BIG = 9999999.0


def _mine_kernel(e1_ref, e2_ref, l1_ref, l2_ref, out_ref, n1_ref, e1b_ref):
    j = pl.program_id(0)
    bm = e1_ref.shape[0]

    @pl.when(j == 0)
    def _cache_e1():
        e1 = e1_ref[...]
        n1 = jnp.sqrt(jnp.sum(e1 * e1, axis=1, keepdims=True)) + 1e-12
        n1_ref[0, :] = n1.reshape(bm)
        e1b_ref[...] = e1.astype(jnp.bfloat16)

    e2 = e2_ref[...]
    inv2 = 1.0 / (jnp.sqrt(jnp.sum(e2 * e2, axis=1, keepdims=True)) + 1e-12)
    e2n = (e2 * inv2).astype(jnp.bfloat16)

    raw = jax.lax.dot_general(
        e2n, e1b_ref[...], (((1,), (1,)), ((), ())),
        preferred_element_type=jnp.float32,
    )

    norm1 = n1_ref[0, :].reshape(1, bm)
    mask = l2_ref[...] == l1_ref[...]
    p_sent = _BIG * norm1
    p_tile = jnp.min(jnp.where(mask, raw, p_sent), axis=0) / norm1[0, :]
    n_tile = jnp.max(jnp.where(mask, -p_sent, raw), axis=0) / norm1[0, :]

    @pl.when(j == 0)
    def _init():
        out_ref[0, :] = p_tile
        out_ref[1, :] = n_tile

    @pl.when(j != 0)
    def _fold():
        out_ref[0, :] = jnp.minimum(out_ref[0, :], p_tile)
        out_ref[1, :] = jnp.maximum(out_ref[1, :], n_tile)


@functools.partial(jax.jit, static_argnames=("bn",))
def _mine(emb1, emb2, label1, label2, bn=1024):
    m, k = emb1.shape
    n = emb2.shape[0]
    l1 = label1.reshape(1, m)
    l2 = label2.reshape(n, 1)
    grid = (n // bn,)
    return pl.pallas_call(
        _mine_kernel,
        grid=grid,
        in_specs=[
            pl.BlockSpec((m, k), lambda j: (0, 0)),
            pl.BlockSpec((bn, k), lambda j: (j, 0)),
            pl.BlockSpec((1, m), lambda j: (0, 0)),
            pl.BlockSpec((bn, 1), lambda j: (j, 0)),
        ],
        out_specs=pl.BlockSpec((2, m), lambda j: (0, 0)),
        out_shape=jax.ShapeDtypeStruct((2, m), jnp.float32),
        scratch_shapes=[
            pltpu.VMEM((1, m), jnp.float32),
            pltpu.VMEM((m, k), jnp.bfloat16),
        ],
        compiler_params=pltpu.CompilerParams(
            dimension_semantics=("arbitrary",),
        ),
    )(emb1, emb2, l1, l2)


def kernel(emb1, emb2, label1, label2):
    return _mine(emb1, emb2, label1, label2)
